# baseline (speedup 1.0000x reference)
import functools

import jax
import jax.numpy as jnp
import numpy as np
from jax import lax
from jax.experimental import pallas as pl
from jax.experimental.pallas import tpu as pltpu
from jax.experimental.pallas import tpu_sc as plsc

N = 10000
E = 320000
D = 128
H = 8
DH = D // H
NLAYERS = 4
ACCW = D + 16

NC = 2
NS = 16
NW = NC * NS
EPW = E // NW
CH = 80
NCHUNK = EPW // CH

BE = 2000
GE = E // BE
BN = 1000
GN = N // BN

@functools.cache
def _sc_mesh():
    return plsc.VectorSubcoreMesh(core_axis_name="c", subcore_axis_name="s")

_f32 = jnp.float32


def _mm(a, b):
    return jax.lax.dot_general(a, b, (((1,), (0,)), ((), ())),
                               preferred_element_type=_f32)


def _mm_bf(a, b):
    return jax.lax.dot_general(a.astype(jnp.bfloat16), b.astype(jnp.bfloat16),
                               (((1,), (0,)), ((), ())),
                               preferred_element_type=_f32)


def _ln_rows(x, s, b):
    m = jnp.mean(x, axis=-1, keepdims=True)
    v = jnp.mean((x - m) * (x - m), axis=-1, keepdims=True)
    return (x - m) / jnp.sqrt(v + 1e-5) * s + b


def _sc_gather_kq(k_tab, q_tab, src, dst):
    out_t = jax.ShapeDtypeStruct((E, D), _f32)

    @functools.partial(
        pl.kernel,
        out_type=out_t,
        mesh=_sc_mesh(),
        scratch_types=[
            pltpu.VMEM((EPW,), jnp.int32),
            pltpu.VMEM((EPW,), jnp.int32),
            pltpu.VMEM((CH, D), _f32), pltpu.VMEM((CH, D), _f32),
            pltpu.VMEM((CH, D), _f32), pltpu.VMEM((CH, D), _f32),
            pltpu.VMEM((CH, D), _f32), pltpu.VMEM((CH, D), _f32),
            pltpu.SemaphoreType.DMA, pltpu.SemaphoreType.DMA,
            pltpu.SemaphoreType.DMA, pltpu.SemaphoreType.DMA,
            pltpu.SemaphoreType.DMA, pltpu.SemaphoreType.DMA,
        ],
    )
    def run(k_hbm, q_hbm, src_hbm, dst_hbm, kq_hbm,
            sia, dia, kb0, kb1, qb0, qb1, ob0, ob1,
            gk0, gk1, gq0, gq1, so0, so1):
        wid = lax.axis_index("s") * NC + lax.axis_index("c")
        base = wid * EPW
        pltpu.sync_copy(src_hbm.at[pl.ds(base, EPW)], sia)
        pltpu.sync_copy(dst_hbm.at[pl.ds(base, EPW)], dia)

        def issue(c, kb, qb, gk, gq):
            loc = c * CH
            pltpu.async_copy(k_hbm.at[sia.at[pl.ds(loc, CH)]], kb, gk)
            pltpu.async_copy(q_hbm.at[dia.at[pl.ds(loc, CH)]], qb, gq)

        def finish(c, kb, qb, ob, gk, gq, so, drain):
            loc = c * CH
            pltpu.make_async_copy(k_hbm.at[sia.at[pl.ds(loc, CH)]], kb, gk).wait()
            pltpu.make_async_copy(q_hbm.at[dia.at[pl.ds(loc, CH)]], qb, gq).wait()

            @pl.when(drain)
            def _():
                pltpu.make_async_copy(ob, kq_hbm.at[pl.ds(base, CH)], so).wait()

            @pl.loop(0, CH)
            def _e(i):
                for j in range(0, D, 16):
                    ob[i, pl.ds(j, 16)] = kb[i, pl.ds(j, 16)] * qb[i, pl.ds(j, 16)]

            pltpu.async_copy(ob, kq_hbm.at[pl.ds(base + loc, CH)], so)

        issue(0, kb0, qb0, gk0, gq0)

        @pl.loop(0, (NCHUNK - 1) // 2)
        def _steady(i):
            c = i * 2
            issue(c + 1, kb1, qb1, gk1, gq1)
            finish(c, kb0, qb0, ob0, gk0, gq0, so0, i > 0)
            issue(c + 2, kb0, qb0, gk0, gq0)
            finish(c + 1, kb1, qb1, ob1, gk1, gq1, so1, i > 0)

        finish(NCHUNK - 1, kb0, qb0, ob0, gk0, gq0, so0, True)
        pltpu.make_async_copy(ob0, kq_hbm.at[pl.ds(base, CH)], so0).wait()
        pltpu.make_async_copy(ob1, kq_hbm.at[pl.ds(base, CH)], so1).wait()

    return run(k_tab, q_tab, src, dst)


SCH = 40
SNCHUNK = EPW // SCH
SIB = 50
SNB = SNCHUNK // SIB


def _sc_scatter(v_tab, src3, dst3, sc_cat, zrows):
    out_t = jax.ShapeDtypeStruct((NC, N, ACCW), _f32)

    @functools.partial(
        pl.kernel,
        out_type=out_t,
        mesh=_sc_mesh(),
        compiler_params=pltpu.CompilerParams(use_tc_tiling_on_sc=False),
        scratch_types=[
            pltpu.VMEM((SIB, SCH), jnp.int32), pltpu.VMEM((SIB, SCH), jnp.int32),
            pltpu.VMEM((SCH, D), _f32), pltpu.VMEM((SCH, D), _f32),
            pltpu.VMEM((SCH, ACCW), _f32), pltpu.VMEM((SCH, ACCW), _f32),
            pltpu.VMEM((SCH, ACCW), _f32), pltpu.VMEM((SCH, ACCW), _f32),
            pltpu.VMEM_SHARED((N, ACCW), _f32),
            pltpu.SemaphoreType.DMA, pltpu.SemaphoreType.DMA,
            pltpu.SemaphoreType.DMA, pltpu.SemaphoreType.DMA,
            pltpu.SemaphoreType.DMA, pltpu.SemaphoreType.DMA,
        ],
    )
    def run(v_hbm, src_hbm, dst_hbm, sc_hbm, z_hbm, out_hbm,
            sblk, dblk, vb0, vb1, scb0, scb1, mb0, mb1, acc,
            gv0, gv1, gs0, gs1, sa0, sa1):
        cid = lax.axis_index("c")
        sid = lax.axis_index("s")
        wid = sid * NC + cid

        @pl.when(sid < 15)
        def _():
            pltpu.sync_copy(z_hbm, acc.at[pl.ds(sid * 640, 640)])

        @pl.when(sid == 15)
        def _():
            pltpu.sync_copy(z_hbm.at[pl.ds(0, 400)], acc.at[pl.ds(9600, 400)])

        base = wid * EPW
        plsc.subcore_barrier()

        def issue(b, j, vb, scb, gv, gs):
            off = base + (b * SIB + j) * SCH
            pltpu.async_copy(v_hbm.at[sblk.at[j]], vb, gv)
            pltpu.async_copy(sc_hbm.at[pl.ds(off, SCH)], scb, gs)

        def finish(b, j, vb, scb, mb, gv, gs, sa, drain):
            off = base + (b * SIB + j) * SCH
            pltpu.make_async_copy(v_hbm.at[sblk.at[j]], vb, gv).wait()
            pltpu.make_async_copy(sc_hbm.at[pl.ds(off, SCH)], scb, gs).wait()

            @pl.when(drain)
            def _():
                pltpu.make_async_copy(mb, acc.at[pl.ds(0, SCH)], sa).wait()

            @pl.loop(0, SCH)
            def _edge(i):
                for jj in range(0, D, 16):
                    mb[i, pl.ds(jj, 16)] = vb[i, pl.ds(jj, 16)] * scb[i, pl.ds(jj, 16)]
                mb[i, pl.ds(D, 16)] = scb[i, pl.ds(D, 16)]

            pltpu.async_copy(mb, acc.at[dblk.at[j]], sa, add=True)

        @pl.loop(0, SNB)
        def _block(b):
            pltpu.sync_copy(src_hbm.at[wid, pl.ds(b * SIB, SIB)], sblk)
            pltpu.sync_copy(dst_hbm.at[wid, pl.ds(b * SIB, SIB)], dblk)
            issue(b, 0, vb0, scb0, gv0, gs0)
            issue(b, 1, vb1, scb1, gv1, gs1)

            @pl.loop(0, SIB // 2 - 1)
            def _steady(i):
                c = i * 2
                finish(b, c, vb0, scb0, mb0, gv0, gs0, sa0, i > 0)
                issue(b, c + 2, vb0, scb0, gv0, gs0)
                finish(b, c + 1, vb1, scb1, mb1, gv1, gs1, sa1, i > 0)
                issue(b, c + 3, vb1, scb1, gv1, gs1)

            finish(b, SIB - 2, vb0, scb0, mb0, gv0, gs0, sa0, True)
            finish(b, SIB - 1, vb1, scb1, mb1, gv1, gs1, sa1, True)
            pltpu.make_async_copy(mb0, acc.at[pl.ds(0, SCH)], sa0).wait()
            pltpu.make_async_copy(mb1, acc.at[pl.ds(0, SCH)], sa1).wait()

        plsc.subcore_barrier()

        @pl.when(sid < 15)
        def _():
            pltpu.sync_copy(acc.at[pl.ds(sid * 640, 640)],
                            out_hbm.at[cid, pl.ds(sid * 640, 640)])

        @pl.when(sid == 15)
        def _():
            pltpu.sync_copy(acc.at[pl.ds(9600, 400)],
                            out_hbm.at[cid, pl.ds(9600, 400)])

    return run(v_tab, src3, dst3, sc_cat, zrows)


def _input_body(h_ref, pe_ref, Wh_ref, bh_ref, Wp_ref, bp_ref,
                Qw_ref, Kw_ref, Vw_ref,
                hh_ref, q_ref, k_ref, v_ref):
    hh = _mm(h_ref[...], Wh_ref[...]) + bh_ref[...]
    hh = hh + _mm(pe_ref[...], Wp_ref[...]) + bp_ref[...]
    hh_ref[...] = hh
    q_ref[...] = _mm(hh, Qw_ref[...])
    k_ref[...] = _mm(hh, Kw_ref[...])
    v_ref[...] = _mm(hh, Vw_ref[...])


def _edge_core(ef_blk, kq_ref, Epr_ref, Hm_ref, Ecat_ref):
    pe = _mm(ef_blk, Epr_ref[...])
    s = kq_ref[...] * pe * 0.25
    ssum = _mm(s, Hm_ref[...])
    sce = jnp.exp(jnp.clip(ssum, -5.0, 5.0))
    return s, _mm(sce, Ecat_ref[...])


def _edge_first_body(e_ref, kq_ref, Et_ref, Epr_ref, Hm_ref, Ecat_ref,
                     OeW_ref, Oeb_ref, l1s_ref, l1b_ref,
                     e2_ref, sc_ref):
    ev = e_ref[...]
    ef_blk = jnp.zeros((ev.shape[0], D), _f32)
    for kb in range(4):
        ef_blk = ef_blk + jnp.where(ev == float(kb), 1.0, 0.0) * Et_ref[kb:kb + 1, :]
    s, sc_cat = _edge_core(ef_blk, kq_ref, Epr_ref, Hm_ref, Ecat_ref)
    sc_ref[...] = sc_cat
    e2 = _mm_bf(s, OeW_ref[...]) + Oeb_ref[...] + ef_blk
    e2_ref[...] = _ln_rows(e2, l1s_ref[...], l1b_ref[...])


def _edge_mid_body(ef_ref, kq_ref, Epr_ref, Hm_ref, Ecat_ref,
                   OeW_ref, Oeb_ref, l1s_ref, l1b_ref,
                   e2_ref, sc_ref):
    ef_blk = ef_ref[...]
    s, sc_cat = _edge_core(ef_blk, kq_ref, Epr_ref, Hm_ref, Ecat_ref)
    sc_ref[...] = sc_cat
    e2 = _mm_bf(s, OeW_ref[...]) + Oeb_ref[...] + ef_blk
    e2_ref[...] = _ln_rows(e2, l1s_ref[...], l1b_ref[...])


def _edge_ffn_body(e2_ref, W1_ref, b1_ref, W2_ref, b2_ref, l2s_ref, l2b_ref,
                   efo_ref):
    e2 = e2_ref[...]
    f = jnp.maximum(_mm_bf(e2, W1_ref[...]) + b1_ref[...], 0.0)
    f = _mm_bf(f, W2_ref[...]) + b2_ref[...]
    efo_ref[...] = _ln_rows(e2 + f, l2s_ref[...], l2b_ref[...])


def _edge_last_body(ef_ref, kq_ref, Epr_ref, Hm_ref, Ecat_ref, sc_ref):
    _, sc_cat = _edge_core(ef_ref[...], kq_ref, Epr_ref, Hm_ref, Ecat_ref)
    sc_ref[...] = sc_cat


def _node_head(a0_ref, a1_ref, hin_ref, Zex_ref, OW_ref, Ob_ref,
               l1s_ref, l1b_ref, W1_ref, b1_ref, W2_ref, b2_ref,
               l2s_ref, l2b_ref):
    acc = a0_ref[...] + a1_ref[...]
    wv = acc[:, :D]
    z = _mm(acc[:, D:], Zex_ref[...])
    hatt = wv / (z + 1e-6)
    h2 = _mm(hatt, OW_ref[...]) + Ob_ref[...] + hin_ref[...]
    h2 = _ln_rows(h2, l1s_ref[...], l1b_ref[...])
    f = jnp.maximum(_mm(h2, W1_ref[...]) + b1_ref[...], 0.0)
    f = _mm(f, W2_ref[...]) + b2_ref[...]
    return _ln_rows(h2 + f, l2s_ref[...], l2b_ref[...])


def _node_mid_body(a0_ref, a1_ref, hin_ref, Zex_ref, OW_ref, Ob_ref,
                   l1s_ref, l1b_ref, W1_ref, b1_ref, W2_ref, b2_ref,
                   l2s_ref, l2b_ref, Qw_ref, Kw_ref, Vw_ref,
                   ho_ref, q_ref, k_ref, v_ref):
    hn = _node_head(a0_ref, a1_ref, hin_ref, Zex_ref, OW_ref, Ob_ref,
                    l1s_ref, l1b_ref, W1_ref, b1_ref, W2_ref, b2_ref,
                    l2s_ref, l2b_ref)
    ho_ref[...] = hn
    q_ref[...] = _mm(hn, Qw_ref[...])
    k_ref[...] = _mm(hn, Kw_ref[...])
    v_ref[...] = _mm(hn, Vw_ref[...])


def _node_last_body(a0_ref, a1_ref, hin_ref, Zex_ref, OW_ref, Ob_ref,
                    l1s_ref, l1b_ref, W1_ref, b1_ref, W2_ref, b2_ref,
                    l2s_ref, l2b_ref, out_ref):
    hn = _node_head(a0_ref, a1_ref, hin_ref, Zex_ref, OW_ref, Ob_ref,
                    l1s_ref, l1b_ref, W1_ref, b1_ref, W2_ref, b2_ref,
                    l2s_ref, l2b_ref)

    @pl.when(pl.program_id(0) == 0)
    def _():
        out_ref[...] = jnp.zeros_like(out_ref)

    out_ref[...] += jnp.sum(hn, axis=0, keepdims=True) * (1.0 / N)


def _full(shape):
    return pl.BlockSpec(shape, lambda i: tuple(0 for _ in shape))


def _rows(bs, w):
    return pl.BlockSpec((bs, w), lambda i: (i, 0))


def _input_call(h40, hpe, Wh, bh, Wp, bp, Qw, Kw, Vw):
    o = jax.ShapeDtypeStruct((N, D), _f32)
    return pl.pallas_call(
        _input_body,
        grid=(GN,),
        in_specs=[_rows(BN, 40), _rows(BN, PE8), _full((40, D)), _full((1, D)),
                  _full((PE8, D)), _full((1, D)),
                  _full((D, D)), _full((D, D)), _full((D, D))],
        out_specs=[_rows(BN, D)] * 4,
        out_shape=[o, o, o, o],
    )(h40, hpe, Wh, bh, Wp, bp, Qw, Kw, Vw)


def _edge_weight_specs():
    return [_full((D, D)), _full((D, H)), _full((H, ACCW)),
            _full((D, D)), _full((1, D)), _full((1, D)), _full((1, D))]


def _edge_first_call(e_f, kq, Et, Epr, Hm, Ecat, OeW, Oeb, l1s, l1b):
    return pl.pallas_call(
        _edge_first_body,
        grid=(GE,),
        in_specs=[_rows(BE, 1), _rows(BE, D), _full((8, D))]
        + _edge_weight_specs(),
        out_specs=[_rows(BE, D), _rows(BE, ACCW)],
        out_shape=[jax.ShapeDtypeStruct((E, D), _f32),
                   jax.ShapeDtypeStruct((E, ACCW), _f32)],
    )(e_f, kq, Et, Epr, Hm, Ecat, OeW, Oeb, l1s, l1b)


def _edge_mid_call(ef, kq, Epr, Hm, Ecat, OeW, Oeb, l1s, l1b):
    return pl.pallas_call(
        _edge_mid_body,
        grid=(GE,),
        in_specs=[_rows(BE, D), _rows(BE, D)]
        + _edge_weight_specs(),
        out_specs=[_rows(BE, D), _rows(BE, ACCW)],
        out_shape=[jax.ShapeDtypeStruct((E, D), _f32),
                   jax.ShapeDtypeStruct((E, ACCW), _f32)],
    )(ef, kq, Epr, Hm, Ecat, OeW, Oeb, l1s, l1b)


def _edge_ffn_call(e2, W1, b1, W2, b2, l2s, l2b):
    return pl.pallas_call(
        _edge_ffn_body,
        grid=(GE,),
        in_specs=[_rows(BE, D), _full((D, 2 * D)), _full((1, 2 * D)),
                  _full((2 * D, D)), _full((1, D)), _full((1, D)),
                  _full((1, D))],
        out_specs=[_rows(BE, D)],
        out_shape=[jax.ShapeDtypeStruct((E, D), _f32)],
    )(e2, W1, b1, W2, b2, l2s, l2b)[0]


def _edge_last_call(ef, kq, Epr, Hm, Ecat):
    return pl.pallas_call(
        _edge_last_body,
        grid=(GE,),
        in_specs=[_rows(BE, D), _rows(BE, D),
                  _full((D, D)), _full((D, H)), _full((H, ACCW))],
        out_specs=[_rows(BE, ACCW)],
        out_shape=[jax.ShapeDtypeStruct((E, ACCW), _f32)],
    )(ef, kq, Epr, Hm, Ecat)[0]


def _node_weight_specs():
    return [_full((16, D)), _full((D, D)), _full((1, D)), _full((1, D)),
            _full((1, D)), _full((D, 2 * D)), _full((1, 2 * D)),
            _full((2 * D, D)), _full((1, D)), _full((1, D)), _full((1, D))]


def _node_mid_call(a0, a1, hin, Zex, OW, Ob, l1s, l1b, W1, b1, W2, b2,
                   l2s, l2b, Qw, Kw, Vw):
    o = jax.ShapeDtypeStruct((N, D), _f32)
    return pl.pallas_call(
        _node_mid_body,
        grid=(GN,),
        in_specs=[_rows(BN, ACCW), _rows(BN, ACCW), _rows(BN, D)]
        + _node_weight_specs()
        + [_full((D, D)), _full((D, D)), _full((D, D))],
        out_specs=[_rows(BN, D)] * 4,
        out_shape=[o, o, o, o],
    )(a0, a1, hin, Zex, OW, Ob, l1s, l1b, W1, b1, W2, b2, l2s, l2b, Qw, Kw, Vw)


def _node_last_call(a0, a1, hin, Zex, OW, Ob, l1s, l1b, W1, b1, W2, b2,
                    l2s, l2b):
    return pl.pallas_call(
        _node_last_body,
        grid=(GN,),
        in_specs=[_rows(BN, ACCW), _rows(BN, ACCW), _rows(BN, D)]
        + _node_weight_specs(),
        out_specs=[_full((1, D))],
        out_shape=[jax.ShapeDtypeStruct((1, D), _f32)],
    )(a0, a1, hin, Zex, OW, Ob, l1s, l1b, W1, b1, W2, b2, l2s, l2b)[0]


PE8 = 8

_HM = np.zeros((D, H), np.float32)
for _d in range(D):
    _HM[_d, _d // DH] = 1.0
_ECAT = np.zeros((H, ACCW), np.float32)
for _h in range(H):
    _ECAT[_h, _h * DH:(_h + 1) * DH] = 1.0
    _ECAT[_h, D + _h] = 1.0
_ZEX = np.zeros((16, D), np.float32)
for _h in range(H):
    _ZEX[_h, _h * DH:(_h + 1) * DH] = 1.0


def kernel(h, h_lap_pos_enc, edge_index, e, params):
    src = edge_index[0].astype(jnp.int32)
    dst = edge_index[1].astype(jnp.int32)
    src3 = src.reshape(NW, SNCHUNK, SCH)
    dst3 = dst.reshape(NW, SNCHUNK, SCH)
    e_f = e.astype(_f32).reshape(E, 1)
    zrows = jnp.zeros((640, ACCW), _f32)
    etab8 = jnp.pad(params['E_tab'], ((0, 4), (0, 0)))
    h40 = jnp.pad(h, ((0, 0), (0, 5)))
    W40 = jnp.pad(params['W_h2'], ((0, 5), (0, 0)))
    Hm = jnp.asarray(_HM)
    Ecat = jnp.asarray(_ECAT)
    Zex = jnp.asarray(_ZEX)

    def r(x):
        return x.reshape(1, -1)

    p0 = params['layers'][0]
    hh, q, k, v = _input_call(h40, h_lap_pos_enc, W40, r(params['b_h2']),
                              params['W_pe'], r(params['b_pe']),
                              p0['Q'], p0['K'], p0['V'])
    ef = None
    for l in range(NLAYERS):
        p = params['layers'][l]
        kq = _sc_gather_kq(k, q, src, dst)
        if l == 0:
            e2, sc_cat = _edge_first_call(
                e_f, kq, etab8, p['Epr'], Hm, Ecat,
                p['Oe_W'], r(p['Oe_b']), r(p['ln1e_s']), r(p['ln1e_b']))
        elif l < NLAYERS - 1:
            e2, sc_cat = _edge_mid_call(
                ef, kq, p['Epr'], Hm, Ecat,
                p['Oe_W'], r(p['Oe_b']), r(p['ln1e_s']), r(p['ln1e_b']))
        else:
            sc_cat = _edge_last_call(ef, kq, p['Epr'], Hm, Ecat)
        acc = _sc_scatter(v, src3, dst3, sc_cat, zrows)
        if l < NLAYERS - 1:
            ef = _edge_ffn_call(e2, p['ffe_W1'], r(p['ffe_b1']),
                                p['ffe_W2'], r(p['ffe_b2']),
                                r(p['ln2e_s']), r(p['ln2e_b']))
        if l < NLAYERS - 1:
            pn = params['layers'][l + 1]
            hh, q, k, v = _node_mid_call(
                acc[0], acc[1], hh, Zex, p['O_W'], r(p['O_b']),
                r(p['ln1h_s']), r(p['ln1h_b']),
                p['ffh_W1'], r(p['ffh_b1']), p['ffh_W2'], r(p['ffh_b2']),
                r(p['ln2h_s']), r(p['ln2h_b']), pn['Q'], pn['K'], pn['V'])
        else:
            hg = _node_last_call(
                acc[0], acc[1], hh, Zex, p['O_W'], r(p['O_b']),
                r(p['ln1h_s']), r(p['ln1h_b']),
                p['ffh_W1'], r(p['ffh_b1']), p['ffh_W2'], r(p['ffh_b2']),
                r(p['ln2h_s']), r(p['ln2h_b']))
    return hg


# speedup vs baseline: 1.2534x; 1.2534x over previous
import functools

import jax
import jax.numpy as jnp
import numpy as np
from jax import lax
from jax.experimental import pallas as pl
from jax.experimental.pallas import tpu as pltpu
from jax.experimental.pallas import tpu_sc as plsc

N = 10000
E = 320000
D = 128
H = 8
DH = D // H
NLAYERS = 4
ACCW = D + 16

NC = 2
NS = 16
NW = NC * NS
EPW = E // NW
CH = 80
NCHUNK = EPW // CH

BE = 2000
GE = E // BE
BN = 1000
GN = N // BN

@functools.cache
def _sc_mesh():
    return plsc.VectorSubcoreMesh(core_axis_name="c", subcore_axis_name="s")

_f32 = jnp.float32


def _mm(a, b):
    return jax.lax.dot_general(a, b, (((1,), (0,)), ((), ())),
                               preferred_element_type=_f32)


def _mm_bf(a, b):
    return jax.lax.dot_general(a.astype(jnp.bfloat16), b.astype(jnp.bfloat16),
                               (((1,), (0,)), ((), ())),
                               preferred_element_type=_f32)


def _ln_rows(x, s, b):
    m = jnp.mean(x, axis=-1, keepdims=True)
    v = jnp.mean((x - m) * (x - m), axis=-1, keepdims=True)
    return (x - m) / jnp.sqrt(v + 1e-5) * s + b


def _sc_gather_kq(k_tab, q_tab, src, dst):
    out_t = jax.ShapeDtypeStruct((E, D), _f32)

    @functools.partial(
        pl.kernel,
        out_type=out_t,
        mesh=_sc_mesh(),
        scratch_types=[
            pltpu.VMEM((EPW,), jnp.int32),
            pltpu.VMEM((EPW,), jnp.int32),
            pltpu.VMEM((CH, D), _f32), pltpu.VMEM((CH, D), _f32),
            pltpu.VMEM((CH, D), _f32), pltpu.VMEM((CH, D), _f32),
            pltpu.VMEM((CH, D), _f32), pltpu.VMEM((CH, D), _f32),
            pltpu.SemaphoreType.DMA, pltpu.SemaphoreType.DMA,
            pltpu.SemaphoreType.DMA, pltpu.SemaphoreType.DMA,
            pltpu.SemaphoreType.DMA, pltpu.SemaphoreType.DMA,
        ],
    )
    def run(k_hbm, q_hbm, src_hbm, dst_hbm, kq_hbm,
            sia, dia, kb0, kb1, qb0, qb1, ob0, ob1,
            gk0, gk1, gq0, gq1, so0, so1):
        wid = lax.axis_index("s") * NC + lax.axis_index("c")
        base = wid * EPW
        pltpu.sync_copy(src_hbm.at[pl.ds(base, EPW)], sia)
        pltpu.sync_copy(dst_hbm.at[pl.ds(base, EPW)], dia)

        def issue(c, kb, qb, gk, gq):
            loc = c * CH
            pltpu.async_copy(k_hbm.at[sia.at[pl.ds(loc, CH)]], kb, gk)
            pltpu.async_copy(q_hbm.at[dia.at[pl.ds(loc, CH)]], qb, gq)

        def finish(c, kb, qb, ob, gk, gq, so, drain):
            loc = c * CH
            pltpu.make_async_copy(k_hbm.at[sia.at[pl.ds(loc, CH)]], kb, gk).wait()
            pltpu.make_async_copy(q_hbm.at[dia.at[pl.ds(loc, CH)]], qb, gq).wait()

            @pl.when(drain)
            def _():
                pltpu.make_async_copy(ob, kq_hbm.at[pl.ds(base, CH)], so).wait()

            @pl.loop(0, CH)
            def _e(i):
                for j in range(0, D, 16):
                    ob[i, pl.ds(j, 16)] = kb[i, pl.ds(j, 16)] * qb[i, pl.ds(j, 16)]

            pltpu.async_copy(ob, kq_hbm.at[pl.ds(base + loc, CH)], so)

        issue(0, kb0, qb0, gk0, gq0)

        @pl.loop(0, (NCHUNK - 1) // 2)
        def _steady(i):
            c = i * 2
            issue(c + 1, kb1, qb1, gk1, gq1)
            finish(c, kb0, qb0, ob0, gk0, gq0, so0, i > 0)
            issue(c + 2, kb0, qb0, gk0, gq0)
            finish(c + 1, kb1, qb1, ob1, gk1, gq1, so1, i > 0)

        finish(NCHUNK - 1, kb0, qb0, ob0, gk0, gq0, so0, True)
        pltpu.make_async_copy(ob0, kq_hbm.at[pl.ds(base, CH)], so0).wait()
        pltpu.make_async_copy(ob1, kq_hbm.at[pl.ds(base, CH)], so1).wait()

    return run(k_tab, q_tab, src, dst)


SCH = 40
SNCHUNK = EPW // SCH
SIB = 50
SNB = SNCHUNK // SIB


def _sc_scatter(v_tab, src3, dst3, sc_cat, zrows):
    out_t = jax.ShapeDtypeStruct((NC, N, ACCW), _f32)

    @functools.partial(
        pl.kernel,
        out_type=out_t,
        mesh=_sc_mesh(),
        compiler_params=pltpu.CompilerParams(use_tc_tiling_on_sc=False),
        scratch_types=[
            pltpu.VMEM((SIB, SCH), jnp.int32), pltpu.VMEM((SIB, SCH), jnp.int32),
            pltpu.VMEM((SCH, D), _f32), pltpu.VMEM((SCH, D), _f32),
            pltpu.VMEM((SCH, 16), _f32), pltpu.VMEM((SCH, 16), _f32),
            pltpu.VMEM((SCH, ACCW), _f32), pltpu.VMEM((SCH, ACCW), _f32),
            pltpu.VMEM_SHARED((N, ACCW), _f32),
            pltpu.SemaphoreType.DMA, pltpu.SemaphoreType.DMA,
            pltpu.SemaphoreType.DMA, pltpu.SemaphoreType.DMA,
            pltpu.SemaphoreType.DMA, pltpu.SemaphoreType.DMA,
        ],
    )
    def run(v_hbm, src_hbm, dst_hbm, sc_hbm, z_hbm, out_hbm,
            sblk, dblk, vb0, vb1, scb0, scb1, mb0, mb1, acc,
            gv0, gv1, gs0, gs1, sa0, sa1):
        cid = lax.axis_index("c")
        sid = lax.axis_index("s")
        wid = sid * NC + cid

        @pl.when(sid < 15)
        def _():
            pltpu.sync_copy(z_hbm, acc.at[pl.ds(sid * 640, 640)])

        @pl.when(sid == 15)
        def _():
            pltpu.sync_copy(z_hbm.at[pl.ds(0, 400)], acc.at[pl.ds(9600, 400)])

        base = wid * EPW
        plsc.subcore_barrier()

        def issue(b, j, vb, scb, gv, gs):
            off = base + (b * SIB + j) * SCH
            pltpu.async_copy(v_hbm.at[sblk.at[j]], vb, gv)
            pltpu.async_copy(sc_hbm.at[pl.ds(off, SCH)], scb, gs)

        def finish(b, j, vb, scb, mb, gv, gs, sa, drain):
            off = base + (b * SIB + j) * SCH
            pltpu.make_async_copy(v_hbm.at[sblk.at[j]], vb, gv).wait()
            pltpu.make_async_copy(sc_hbm.at[pl.ds(off, SCH)], scb, gs).wait()

            @pl.when(drain)
            def _():
                pltpu.make_async_copy(mb, acc.at[pl.ds(0, SCH)], sa).wait()

            @pl.loop(0, SCH)
            def _edge(i):
                sv = scb[i, :]
                for hh in range(H):
                    w = lax.gather(
                        sv, jnp.full((16, 1), hh, jnp.int32),
                        lax.GatherDimensionNumbers(
                            offset_dims=(), collapsed_slice_dims=(0,),
                            start_index_map=(0,)),
                        (1,), mode=lax.GatherScatterMode.PROMISE_IN_BOUNDS)
                    mb[i, pl.ds(hh * DH, DH)] = vb[i, pl.ds(hh * DH, DH)] * w
                mb[i, pl.ds(D, 16)] = sv

            pltpu.async_copy(mb, acc.at[dblk.at[j]], sa, add=True)

        @pl.loop(0, SNB)
        def _block(b):
            pltpu.sync_copy(src_hbm.at[wid, pl.ds(b * SIB, SIB)], sblk)
            pltpu.sync_copy(dst_hbm.at[wid, pl.ds(b * SIB, SIB)], dblk)
            issue(b, 0, vb0, scb0, gv0, gs0)
            issue(b, 1, vb1, scb1, gv1, gs1)

            @pl.loop(0, SIB // 2 - 1)
            def _steady(i):
                c = i * 2
                finish(b, c, vb0, scb0, mb0, gv0, gs0, sa0, i > 0)
                issue(b, c + 2, vb0, scb0, gv0, gs0)
                finish(b, c + 1, vb1, scb1, mb1, gv1, gs1, sa1, i > 0)
                issue(b, c + 3, vb1, scb1, gv1, gs1)

            finish(b, SIB - 2, vb0, scb0, mb0, gv0, gs0, sa0, True)
            finish(b, SIB - 1, vb1, scb1, mb1, gv1, gs1, sa1, True)
            pltpu.make_async_copy(mb0, acc.at[pl.ds(0, SCH)], sa0).wait()
            pltpu.make_async_copy(mb1, acc.at[pl.ds(0, SCH)], sa1).wait()

        plsc.subcore_barrier()

        @pl.when(sid < 15)
        def _():
            pltpu.sync_copy(acc.at[pl.ds(sid * 640, 640)],
                            out_hbm.at[cid, pl.ds(sid * 640, 640)])

        @pl.when(sid == 15)
        def _():
            pltpu.sync_copy(acc.at[pl.ds(9600, 400)],
                            out_hbm.at[cid, pl.ds(9600, 400)])

    return run(v_tab, src3, dst3, sc_cat, zrows)


def _input_body(h_ref, pe_ref, Wh_ref, bh_ref, Wp_ref, bp_ref,
                Qw_ref, Kw_ref, Vw_ref,
                hh_ref, q_ref, k_ref, v_ref):
    hh = _mm(h_ref[...], Wh_ref[...]) + bh_ref[...]
    hh = hh + _mm(pe_ref[...], Wp_ref[...]) + bp_ref[...]
    hh_ref[...] = hh
    q_ref[...] = _mm(hh, Qw_ref[...])
    k_ref[...] = _mm(hh, Kw_ref[...])
    v_ref[...] = _mm(hh, Vw_ref[...])


def _edge_core(ef_blk, kq_ref, Epr_ref, Hm_ref, Ecat_ref):
    pe = _mm(ef_blk, Epr_ref[...])
    s = kq_ref[...] * pe * 0.25
    ssum = _mm(s, Hm_ref[...])
    sce = jnp.exp(jnp.clip(ssum, -5.0, 5.0))
    return s, _mm(sce, Ecat_ref[...])


def _edge_tail(s, ef_blk, OeW_ref, Oeb_ref, l1s_ref, l1b_ref,
               W1_ref, b1_ref, W2_ref, b2_ref, l2s_ref, l2b_ref):
    e2 = _mm_bf(s, OeW_ref[...]) + Oeb_ref[...] + ef_blk
    e2 = _ln_rows(e2, l1s_ref[...], l1b_ref[...])
    f = jnp.maximum(_mm_bf(e2, W1_ref[...]) + b1_ref[...], 0.0)
    f = _mm_bf(f, W2_ref[...]) + b2_ref[...]
    return _ln_rows(e2 + f, l2s_ref[...], l2b_ref[...])


def _edge_first_body(e_ref, kq_ref, Et_ref, Epr_ref, Hm_ref, Ecat_ref,
                     OeW_ref, Oeb_ref, l1s_ref, l1b_ref,
                     W1_ref, b1_ref, W2_ref, b2_ref, l2s_ref, l2b_ref,
                     efo_ref, sc_ref):
    ev = e_ref[...]
    ef_blk = jnp.zeros((ev.shape[0], D), _f32)
    for kb in range(4):
        ef_blk = ef_blk + jnp.where(ev == float(kb), 1.0, 0.0) * Et_ref[kb:kb + 1, :]
    s, sc_cat = _edge_core(ef_blk, kq_ref, Epr_ref, Hm_ref, Ecat_ref)
    sc_ref[...] = sc_cat
    efo_ref[...] = _edge_tail(s, ef_blk, OeW_ref, Oeb_ref, l1s_ref, l1b_ref,
                              W1_ref, b1_ref, W2_ref, b2_ref, l2s_ref, l2b_ref)


def _edge_mid_body(ef_ref, kq_ref, Epr_ref, Hm_ref, Ecat_ref,
                   OeW_ref, Oeb_ref, l1s_ref, l1b_ref,
                   W1_ref, b1_ref, W2_ref, b2_ref, l2s_ref, l2b_ref,
                   efo_ref, sc_ref):
    ef_blk = ef_ref[...]
    s, sc_cat = _edge_core(ef_blk, kq_ref, Epr_ref, Hm_ref, Ecat_ref)
    sc_ref[...] = sc_cat
    efo_ref[...] = _edge_tail(s, ef_blk, OeW_ref, Oeb_ref, l1s_ref, l1b_ref,
                              W1_ref, b1_ref, W2_ref, b2_ref, l2s_ref, l2b_ref)


def _edge_last_body(ef_ref, kq_ref, Epr_ref, Hm_ref, Ecat_ref, sc_ref):
    _, sc_cat = _edge_core(ef_ref[...], kq_ref, Epr_ref, Hm_ref, Ecat_ref)
    sc_ref[...] = sc_cat


def _node_head(a0_ref, a1_ref, hin_ref, Zex_ref, OW_ref, Ob_ref,
               l1s_ref, l1b_ref, W1_ref, b1_ref, W2_ref, b2_ref,
               l2s_ref, l2b_ref):
    acc = a0_ref[...] + a1_ref[...]
    wv = acc[:, :D]
    z = _mm(acc[:, D:], Zex_ref[...])
    hatt = wv / (z + 1e-6)
    h2 = _mm(hatt, OW_ref[...]) + Ob_ref[...] + hin_ref[...]
    h2 = _ln_rows(h2, l1s_ref[...], l1b_ref[...])
    f = jnp.maximum(_mm(h2, W1_ref[...]) + b1_ref[...], 0.0)
    f = _mm(f, W2_ref[...]) + b2_ref[...]
    return _ln_rows(h2 + f, l2s_ref[...], l2b_ref[...])


def _node_mid_body(a0_ref, a1_ref, hin_ref, Zex_ref, OW_ref, Ob_ref,
                   l1s_ref, l1b_ref, W1_ref, b1_ref, W2_ref, b2_ref,
                   l2s_ref, l2b_ref, Qw_ref, Kw_ref, Vw_ref,
                   ho_ref, q_ref, k_ref, v_ref):
    hn = _node_head(a0_ref, a1_ref, hin_ref, Zex_ref, OW_ref, Ob_ref,
                    l1s_ref, l1b_ref, W1_ref, b1_ref, W2_ref, b2_ref,
                    l2s_ref, l2b_ref)
    ho_ref[...] = hn
    q_ref[...] = _mm(hn, Qw_ref[...])
    k_ref[...] = _mm(hn, Kw_ref[...])
    v_ref[...] = _mm(hn, Vw_ref[...])


def _node_last_body(a0_ref, a1_ref, hin_ref, Zex_ref, OW_ref, Ob_ref,
                    l1s_ref, l1b_ref, W1_ref, b1_ref, W2_ref, b2_ref,
                    l2s_ref, l2b_ref, out_ref):
    hn = _node_head(a0_ref, a1_ref, hin_ref, Zex_ref, OW_ref, Ob_ref,
                    l1s_ref, l1b_ref, W1_ref, b1_ref, W2_ref, b2_ref,
                    l2s_ref, l2b_ref)

    @pl.when(pl.program_id(0) == 0)
    def _():
        out_ref[...] = jnp.zeros_like(out_ref)

    out_ref[...] += jnp.sum(hn, axis=0, keepdims=True) * (1.0 / N)


def _full(shape):
    return pl.BlockSpec(shape, lambda i: tuple(0 for _ in shape))


def _rows(bs, w):
    return pl.BlockSpec((bs, w), lambda i: (i, 0))


def _input_call(h40, hpe, Wh, bh, Wp, bp, Qw, Kw, Vw):
    o = jax.ShapeDtypeStruct((N, D), _f32)
    return pl.pallas_call(
        _input_body,
        grid=(GN,),
        in_specs=[_rows(BN, 40), _rows(BN, PE8), _full((40, D)), _full((1, D)),
                  _full((PE8, D)), _full((1, D)),
                  _full((D, D)), _full((D, D)), _full((D, D))],
        out_specs=[_rows(BN, D)] * 4,
        out_shape=[o, o, o, o],
    )(h40, hpe, Wh, bh, Wp, bp, Qw, Kw, Vw)


def _edge_weight_specs():
    return [_full((D, D)), _full((D, H)), _full((H, 16)),
            _full((D, D)), _full((1, D)), _full((1, D)), _full((1, D)),
            _full((D, 2 * D)), _full((1, 2 * D)), _full((2 * D, D)),
            _full((1, D)), _full((1, D)), _full((1, D))]


def _edge_first_call(e_f, kq, Et, Epr, Hm, Ecat, OeW, Oeb, l1s, l1b,
                     W1, b1, W2, b2, l2s, l2b):
    return pl.pallas_call(
        _edge_first_body,
        grid=(GE,),
        in_specs=[_rows(BE, 1), _rows(BE, D), _full((8, D))]
        + _edge_weight_specs(),
        out_specs=[_rows(BE, D), _rows(BE, 16)],
        out_shape=[jax.ShapeDtypeStruct((E, D), _f32),
                   jax.ShapeDtypeStruct((E, 16), _f32)],
    )(e_f, kq, Et, Epr, Hm, Ecat, OeW, Oeb, l1s, l1b, W1, b1, W2, b2,
      l2s, l2b)


def _edge_mid_call(ef, kq, Epr, Hm, Ecat, OeW, Oeb, l1s, l1b,
                   W1, b1, W2, b2, l2s, l2b):
    return pl.pallas_call(
        _edge_mid_body,
        grid=(GE,),
        in_specs=[_rows(BE, D), _rows(BE, D)]
        + _edge_weight_specs(),
        out_specs=[_rows(BE, D), _rows(BE, 16)],
        out_shape=[jax.ShapeDtypeStruct((E, D), _f32),
                   jax.ShapeDtypeStruct((E, 16), _f32)],
    )(ef, kq, Epr, Hm, Ecat, OeW, Oeb, l1s, l1b, W1, b1, W2, b2, l2s, l2b)


def _edge_last_call(ef, kq, Epr, Hm, Ecat):
    return pl.pallas_call(
        _edge_last_body,
        grid=(GE,),
        in_specs=[_rows(BE, D), _rows(BE, D),
                  _full((D, D)), _full((D, H)), _full((H, 16))],
        out_specs=[_rows(BE, 16)],
        out_shape=[jax.ShapeDtypeStruct((E, 16), _f32)],
    )(ef, kq, Epr, Hm, Ecat)[0]


def _node_weight_specs():
    return [_full((16, D)), _full((D, D)), _full((1, D)), _full((1, D)),
            _full((1, D)), _full((D, 2 * D)), _full((1, 2 * D)),
            _full((2 * D, D)), _full((1, D)), _full((1, D)), _full((1, D))]


def _node_mid_call(a0, a1, hin, Zex, OW, Ob, l1s, l1b, W1, b1, W2, b2,
                   l2s, l2b, Qw, Kw, Vw):
    o = jax.ShapeDtypeStruct((N, D), _f32)
    return pl.pallas_call(
        _node_mid_body,
        grid=(GN,),
        in_specs=[_rows(BN, ACCW), _rows(BN, ACCW), _rows(BN, D)]
        + _node_weight_specs()
        + [_full((D, D)), _full((D, D)), _full((D, D))],
        out_specs=[_rows(BN, D)] * 4,
        out_shape=[o, o, o, o],
    )(a0, a1, hin, Zex, OW, Ob, l1s, l1b, W1, b1, W2, b2, l2s, l2b, Qw, Kw, Vw)


def _node_last_call(a0, a1, hin, Zex, OW, Ob, l1s, l1b, W1, b1, W2, b2,
                    l2s, l2b):
    return pl.pallas_call(
        _node_last_body,
        grid=(GN,),
        in_specs=[_rows(BN, ACCW), _rows(BN, ACCW), _rows(BN, D)]
        + _node_weight_specs(),
        out_specs=[_full((1, D))],
        out_shape=[jax.ShapeDtypeStruct((1, D), _f32)],
    )(a0, a1, hin, Zex, OW, Ob, l1s, l1b, W1, b1, W2, b2, l2s, l2b)[0]


PE8 = 8

_HM = np.zeros((D, H), np.float32)
for _d in range(D):
    _HM[_d, _d // DH] = 1.0
_ECAT = np.zeros((H, 16), np.float32)
for _h in range(H):
    _ECAT[_h, _h] = 1.0
_ZEX = np.zeros((16, D), np.float32)
for _h in range(H):
    _ZEX[_h, _h * DH:(_h + 1) * DH] = 1.0


def kernel(h, h_lap_pos_enc, edge_index, e, params):
    src = edge_index[0].astype(jnp.int32)
    dst = edge_index[1].astype(jnp.int32)
    src3 = src.reshape(NW, SNCHUNK, SCH)
    dst3 = dst.reshape(NW, SNCHUNK, SCH)
    e_f = e.astype(_f32).reshape(E, 1)
    zrows = jnp.zeros((640, ACCW), _f32)
    etab8 = jnp.pad(params['E_tab'], ((0, 4), (0, 0)))
    h40 = jnp.pad(h, ((0, 0), (0, 5)))
    W40 = jnp.pad(params['W_h2'], ((0, 5), (0, 0)))
    Hm = jnp.asarray(_HM)
    Ecat = jnp.asarray(_ECAT)
    Zex = jnp.asarray(_ZEX)

    def r(x):
        return x.reshape(1, -1)

    p0 = params['layers'][0]
    hh, q, k, v = _input_call(h40, h_lap_pos_enc, W40, r(params['b_h2']),
                              params['W_pe'], r(params['b_pe']),
                              p0['Q'], p0['K'], p0['V'])
    ef = None
    for l in range(NLAYERS):
        p = params['layers'][l]
        kq = _sc_gather_kq(k, q, src, dst)
        if l == 0:
            ef, sc_cat = _edge_first_call(
                e_f, kq, etab8, p['Epr'], Hm, Ecat,
                p['Oe_W'], r(p['Oe_b']), r(p['ln1e_s']), r(p['ln1e_b']),
                p['ffe_W1'], r(p['ffe_b1']), p['ffe_W2'], r(p['ffe_b2']),
                r(p['ln2e_s']), r(p['ln2e_b']))
        elif l < NLAYERS - 1:
            ef, sc_cat = _edge_mid_call(
                ef, kq, p['Epr'], Hm, Ecat,
                p['Oe_W'], r(p['Oe_b']), r(p['ln1e_s']), r(p['ln1e_b']),
                p['ffe_W1'], r(p['ffe_b1']), p['ffe_W2'], r(p['ffe_b2']),
                r(p['ln2e_s']), r(p['ln2e_b']))
        else:
            sc_cat = _edge_last_call(ef, kq, p['Epr'], Hm, Ecat)
        acc = _sc_scatter(v, src3, dst3, sc_cat, zrows)
        if l < NLAYERS - 1:
            pn = params['layers'][l + 1]
            hh, q, k, v = _node_mid_call(
                acc[0], acc[1], hh, Zex, p['O_W'], r(p['O_b']),
                r(p['ln1h_s']), r(p['ln1h_b']),
                p['ffh_W1'], r(p['ffh_b1']), p['ffh_W2'], r(p['ffh_b2']),
                r(p['ln2h_s']), r(p['ln2h_b']), pn['Q'], pn['K'], pn['V'])
        else:
            hg = _node_last_call(
                acc[0], acc[1], hh, Zex, p['O_W'], r(p['O_b']),
                r(p['ln1h_s']), r(p['ln1h_b']),
                p['ffh_W1'], r(p['ffh_b1']), p['ffh_W2'], r(p['ffh_b2']),
                r(p['ln2h_s']), r(p['ln2h_b']))
    return hg
